# Initial kernel scaffold; baseline (speedup 1.0000x reference)
import jax
import jax.numpy as jnp
from jax.experimental import pallas as pl


def kernel(x, pos, edge_index, batch, W0, aS0, aD0, bG0, W1, aS1, aD1, bG1, LW0, Lb0, LW1, Lb1):
    raise NotImplementedError("write your pallas kernel here")


# speedup vs baseline: 1.6934x; 1.6934x over previous
import jax
import jax.numpy as jnp
from jax.experimental import pallas as pl

_N = 10000
_NUM_GRAPHS = 16


def _stage_body(h_ref, w_ref, a_ref, hw_ref, an_ref, self_ref):
    h = h_ref[...]
    hw = jnp.dot(h, w_ref[...], preferred_element_type=jnp.float32)
    an = jnp.dot(hw, a_ref[...], preferred_element_type=jnp.float32)
    an_s = an[:, 0:1]
    an_d = an[:, 1:2]
    t = an_s + an_d
    ex_loop = jnp.exp(jnp.maximum(t, 0.2 * t))
    hw_ref[...] = hw
    an_ref[...] = an
    self_ref[...] = ex_loop * hw


def _stage(h, W, aS, aD):
    a2 = jnp.stack([aS, aD], axis=1)
    hw, an, self_rows = pl.pallas_call(
        _stage_body,
        out_shape=(
            jax.ShapeDtypeStruct((_N, 128), jnp.float32),
            jax.ShapeDtypeStruct((_N, 2), jnp.float32),
            jax.ShapeDtypeStruct((_N, 128), jnp.float32),
        ),
    )(h, W, a2)
    return hw, an, self_rows


def _edge_phase(hw, an, src, dst):
    an_s = an[:, 0]
    an_d = an[:, 1]
    t = an_s[src] + an_d[dst]
    ex = jnp.exp(jnp.maximum(t, 0.2 * t))
    denom = jax.ops.segment_sum(ex, dst, num_segments=_N)
    outn = jax.ops.segment_sum(hw[src] * ex[:, None], dst, num_segments=_N)
    return outn, denom


def _norm_body(outn_ref, selfr_ref, dself_ref, b_ref, o_ref):
    denom = dself_ref[...]
    o_ref[...] = (outn_ref[...] + selfr_ref[...]) / denom + b_ref[...]


def _normalize(outn, edge_denom, self_rows, an, b):
    t = an[:, 0:1] + an[:, 1:2]
    ex_loop = jnp.exp(jnp.maximum(t, 0.2 * t))
    denom = edge_denom[:, None] + ex_loop + 1e-16
    return pl.pallas_call(
        _norm_body,
        out_shape=jax.ShapeDtypeStruct((_N, 128), jnp.float32),
    )(outn, self_rows, denom, b[None, :])


def _head_body(h_ref, bvec_ref, lw0_ref, lb0_ref, lw1_ref, lb1_ref, o_ref):
    h = h_ref[...]
    bcol = bvec_ref[...]
    gids = jax.lax.broadcasted_iota(jnp.int32, (1, _NUM_GRAPHS), 1).astype(jnp.float32)
    mask = (bcol == gids).astype(jnp.float32)
    sums = jax.lax.dot_general(mask, h, (((0,), (0,)), ((), ())),
                               preferred_element_type=jnp.float32)
    counts = jnp.sum(mask, axis=0)[:, None]
    pooled = sums / jnp.maximum(counts, 1.0)
    y = jnp.maximum(pooled @ lw0_ref[...] + lb0_ref[...], 0.0)
    y = jnp.maximum(y @ lw1_ref[...] + lb1_ref[...], 0.0)
    o_ref[...] = y


def _head(h, batch, LW0, Lb0, LW1, Lb1):
    bcol = batch.astype(jnp.float32)[:, None]
    return pl.pallas_call(
        _head_body,
        out_shape=jax.ShapeDtypeStruct((_NUM_GRAPHS, 10), jnp.float32),
    )(h, bcol, LW0, Lb0[None, :], LW1, Lb1[None, :])


def kernel(x, pos, edge_index, batch, W0, aS0, aD0, bG0, W1, aS1, aD1, bG1,
           LW0, Lb0, LW1, Lb1):
    src = edge_index[0]
    dst = edge_index[1]
    h0 = jnp.concatenate([pos, x], axis=1)

    hw0, an0, self0 = _stage(h0, W0, aS0, aD0)
    outn0, den0 = _edge_phase(hw0, an0, src, dst)
    h1 = _normalize(outn0, den0, self0, an0, bG0)

    hw1, an1, self1 = _stage(h1, W1, aS1, aD1)
    outn1, den1 = _edge_phase(hw1, an1, src, dst)
    h2 = _normalize(outn1, den1, self1, an1, bG1)

    return _head(h2, batch, LW0, Lb0, LW1, Lb1)


# speedup vs baseline: 18.2576x; 10.7818x over previous
import dataclasses

import jax
import jax.numpy as jnp
from jax import lax
from jax.experimental import pallas as pl
from jax.experimental.pallas import tpu as pltpu
from jax.experimental.pallas import tpu_sc as plsc

_N = 10000
_E = 320000
_D = 128
_HD = 64
_NUM_GRAPHS = 16

_NBATCH = 250
_BSZ = 80
_NP = 10240
_ROWS_PER_TILE = _NP // 16

_GRID = 8
_BLK = _NP // _GRID


def _stage0_body(h_ref, w_ref, a2_ref, hwl_ref, hwr_ref, an2_ref):
    hw = jnp.dot(h_ref[...], w_ref[...], preferred_element_type=jnp.float32)
    hwl_ref[...] = hw[:, :_HD]
    hwr_ref[...] = hw[:, _HD:]
    an2_ref[...] = jnp.dot(hw, a2_ref[...], preferred_element_type=jnp.float32)


def _stage0(h, W, aS, aD):
    a2 = jnp.stack([aS, aD], axis=1)
    return pl.pallas_call(
        _stage0_body,
        grid=(_GRID,),
        in_specs=[
            pl.BlockSpec((_BLK, _D), lambda i: (i, 0)),
            pl.BlockSpec((_D, _D), lambda i: (0, 0)),
            pl.BlockSpec((_D, 2), lambda i: (0, 0)),
        ],
        out_specs=(
            pl.BlockSpec((_BLK, _HD), lambda i: (i, 0)),
            pl.BlockSpec((_BLK, _HD), lambda i: (i, 0)),
            pl.BlockSpec((_BLK, 2), lambda i: (i, 0)),
        ),
        out_shape=(
            jax.ShapeDtypeStruct((_NP, _HD), jnp.float32),
            jax.ShapeDtypeStruct((_NP, _HD), jnp.float32),
            jax.ShapeDtypeStruct((_NP, 2), jnp.float32),
        ),
    )(h, W, a2)


def _edge_body(hwl_hbm, hwr_hbm, ans_hbm, and_hbm, src_hbm, dst_hbm,
               outp_hbm, denp_hbm,
               ans_v, and_v, src_v, dst_v, ex_v, rows_v, zrows_v,
               out_sh, den_sh):
    cid = lax.axis_index("c")
    sid = lax.axis_index("s")

    pltpu.sync_copy(ans_hbm, ans_v)
    pltpu.sync_copy(and_hbm, and_v)
    pltpu.sync_copy(src_hbm.at[sid], src_v)
    pltpu.sync_copy(dst_hbm.at[sid], dst_v)

    zero16 = jnp.zeros((16,), jnp.float32)

    @pl.loop(0, 40)
    def _fill_zrows(i):
        for f in range(_HD // 16):
            zrows_v[i, pl.ds(16 * f, 16)] = zero16

    rbase = sid * _ROWS_PER_TILE

    @pl.loop(0, 16)
    def _zero_out(i):
        pltpu.sync_copy(zrows_v, out_sh.at[pl.ds(rbase + 40 * i, 40)])

    @pl.when(cid == 0)
    def _zero_den():
        for i in range(10):
            pltpu.sync_copy(zrows_v.at[0],
                            den_sh.at[pl.ds(rbase + 64 * i, 64)])

    plsc.subcore_barrier()

    def _run(hw_hbm, with_den):
        @pl.loop(0, _NBATCH)
        def _batch(j):
            pltpu.sync_copy(hw_hbm.at[src_v.at[j]], rows_v)
            for g in range(5):
                s16 = src_v[j, pl.ds(16 * g, 16)]
                d16 = dst_v[j, pl.ds(16 * g, 16)]
                t = (plsc.load_gather(ans_v, [s16])
                     + plsc.load_gather(and_v, [d16]))
                ex_v[pl.ds(16 * g, 16)] = jnp.exp(jnp.maximum(t, 0.2 * t))
            if with_den:
                pltpu.sync_copy(ex_v, den_sh.at[dst_v.at[j]], add=True)

            @pl.loop(0, _BSZ)
            def _scale(e):
                idx = jnp.full((16,), e, dtype=jnp.int32)
                exs = plsc.load_gather(ex_v, [idx])
                for f in range(_HD // 16):
                    sl = pl.ds(16 * f, 16)
                    rows_v[e, sl] = rows_v[e, sl] * exs

            pltpu.sync_copy(rows_v, out_sh.at[dst_v.at[j]], add=True)

    @pl.when(cid == 0)
    def _run0():
        _run(hwl_hbm, True)

    @pl.when(cid == 1)
    def _run1():
        _run(hwr_hbm, False)

    plsc.subcore_barrier()

    pltpu.sync_copy(out_sh.at[pl.ds(rbase, _ROWS_PER_TILE)],
                    outp_hbm.at[cid, pl.ds(rbase, _ROWS_PER_TILE)])

    @pl.when(cid == 0)
    def _write_den():
        pltpu.sync_copy(den_sh.at[pl.ds(rbase, _ROWS_PER_TILE)],
                        denp_hbm.at[pl.ds(rbase, _ROWS_PER_TILE)])


def _edge_phase(hwl, hwr, ans, andd, src3, dst3):
    mesh = plsc.VectorSubcoreMesh(core_axis_name="c", subcore_axis_name="s")
    cp = pltpu.CompilerParams(needs_layout_passes=False,
                              use_tc_tiling_on_sc=False)
    k = pl.kernel(
        _edge_body,
        compiler_params=cp,
        out_type=(
            jax.ShapeDtypeStruct((2, _NP, _HD), jnp.float32),
            jax.ShapeDtypeStruct((_NP,), jnp.float32),
        ),
        mesh=mesh,
        scratch_types=[
            pltpu.VMEM((_NP,), jnp.float32),
            pltpu.VMEM((_NP,), jnp.float32),
            pltpu.VMEM((_NBATCH, _BSZ), jnp.int32),
            pltpu.VMEM((_NBATCH, _BSZ), jnp.int32),
            pltpu.VMEM((_BSZ,), jnp.float32),
            pltpu.VMEM((_BSZ, _HD), jnp.float32),
            pltpu.VMEM((40, _HD), jnp.float32),
            pltpu.VMEM_SHARED((_NP, _HD), jnp.float32),
            pltpu.VMEM_SHARED((_NP,), jnp.float32),
        ],
    )
    return k(hwl, hwr, ans, andd, src3, dst3)


def _norm_h(outp_ref, denp_ref, hwl_ref, hwr_ref, an2_ref, b_ref):
    an2 = an2_ref[...]
    ex_loop = jnp.exp(jnp.maximum(an2[:, 0:1] + an2[:, 1:2],
                                  0.2 * (an2[:, 0:1] + an2[:, 1:2])))
    hw = jnp.concatenate([hwl_ref[...], hwr_ref[...]], axis=1)
    num = (jnp.concatenate([outp_ref[0], outp_ref[1]], axis=1)
           + ex_loop * hw)
    den = denp_ref[...] + ex_loop + 1e-16
    return num / den + b_ref[...]


def _normstage_body(outp_ref, denp_ref, hwl_ref, hwr_ref, an2_ref, b_ref,
                    w_ref, a2_ref, hwl2_ref, hwr2_ref, an22_ref):
    h = _norm_h(outp_ref, denp_ref, hwl_ref, hwr_ref, an2_ref, b_ref)
    hw2 = jnp.dot(h, w_ref[...], preferred_element_type=jnp.float32)
    hwl2_ref[...] = hw2[:, :_HD]
    hwr2_ref[...] = hw2[:, _HD:]
    an22_ref[...] = jnp.dot(hw2, a2_ref[...],
                            preferred_element_type=jnp.float32)


def _normstage(outp, denp, hwl, hwr, an2, b, W, aS, aD):
    a2 = jnp.stack([aS, aD], axis=1)
    return pl.pallas_call(
        _normstage_body,
        grid=(_GRID,),
        in_specs=[
            pl.BlockSpec((2, _BLK, _HD), lambda i: (0, i, 0)),
            pl.BlockSpec((_BLK, 1), lambda i: (i, 0)),
            pl.BlockSpec((_BLK, _HD), lambda i: (i, 0)),
            pl.BlockSpec((_BLK, _HD), lambda i: (i, 0)),
            pl.BlockSpec((_BLK, 2), lambda i: (i, 0)),
            pl.BlockSpec((1, _D), lambda i: (0, 0)),
            pl.BlockSpec((_D, _D), lambda i: (0, 0)),
            pl.BlockSpec((_D, 2), lambda i: (0, 0)),
        ],
        out_specs=(
            pl.BlockSpec((_BLK, _HD), lambda i: (i, 0)),
            pl.BlockSpec((_BLK, _HD), lambda i: (i, 0)),
            pl.BlockSpec((_BLK, 2), lambda i: (i, 0)),
        ),
        out_shape=(
            jax.ShapeDtypeStruct((_NP, _HD), jnp.float32),
            jax.ShapeDtypeStruct((_NP, _HD), jnp.float32),
            jax.ShapeDtypeStruct((_NP, 2), jnp.float32),
        ),
    )(outp, denp[:, None], hwl, hwr, an2, b[None, :], W, a2)


def _normhead_body(outp_ref, denp_ref, hwl_ref, hwr_ref, an2_ref, b_ref,
                   bvec_ref, lw0_ref, lb0_ref, lw1_ref, lb1_ref, o_ref,
                   sums_ref, counts_ref):
    i = pl.program_id(0)

    @pl.when(i == 0)
    def _init():
        sums_ref[...] = jnp.zeros_like(sums_ref)
        counts_ref[...] = jnp.zeros_like(counts_ref)

    h = _norm_h(outp_ref, denp_ref, hwl_ref, hwr_ref, an2_ref, b_ref)
    bcol = bvec_ref[...]
    gids = jax.lax.broadcasted_iota(
        jnp.int32, (1, _NUM_GRAPHS), 1).astype(jnp.float32)
    mask = (bcol == gids).astype(jnp.float32)
    sums_ref[...] += jax.lax.dot_general(
        mask, h, (((0,), (0,)), ((), ())),
        preferred_element_type=jnp.float32)
    counts_ref[...] += jnp.sum(mask, axis=0)[:, None]

    @pl.when(i == _GRID - 1)
    def _finish():
        pooled = sums_ref[...] / jnp.maximum(counts_ref[...], 1.0)
        y = jnp.maximum(pooled @ lw0_ref[...] + lb0_ref[...], 0.0)
        y = jnp.maximum(y @ lw1_ref[...] + lb1_ref[...], 0.0)
        o_ref[...] = y


def _normhead(outp, denp, hwl, hwr, an2, b, bcol, LW0, Lb0, LW1, Lb1):
    return pl.pallas_call(
        _normhead_body,
        grid=(_GRID,),
        in_specs=[
            pl.BlockSpec((2, _BLK, _HD), lambda i: (0, i, 0)),
            pl.BlockSpec((_BLK, 1), lambda i: (i, 0)),
            pl.BlockSpec((_BLK, _HD), lambda i: (i, 0)),
            pl.BlockSpec((_BLK, _HD), lambda i: (i, 0)),
            pl.BlockSpec((_BLK, 2), lambda i: (i, 0)),
            pl.BlockSpec((1, _D), lambda i: (0, 0)),
            pl.BlockSpec((_BLK, 1), lambda i: (i, 0)),
            pl.BlockSpec((_D, _HD), lambda i: (0, 0)),
            pl.BlockSpec((1, _HD), lambda i: (0, 0)),
            pl.BlockSpec((_HD, 10), lambda i: (0, 0)),
            pl.BlockSpec((1, 10), lambda i: (0, 0)),
        ],
        out_specs=pl.BlockSpec((_NUM_GRAPHS, 10), lambda i: (0, 0)),
        out_shape=jax.ShapeDtypeStruct((_NUM_GRAPHS, 10), jnp.float32),
        scratch_shapes=[
            pltpu.VMEM((_NUM_GRAPHS, _D), jnp.float32),
            pltpu.VMEM((_NUM_GRAPHS, 1), jnp.float32),
        ],
    )(outp, denp[:, None], hwl, hwr, an2, b[None, :],
      bcol, LW0, Lb0[None, :], LW1, Lb1[None, :])


def kernel(x, pos, edge_index, batch, W0, aS0, aD0, bG0, W1, aS1, aD1, bG1,
           LW0, Lb0, LW1, Lb1):
    src3 = edge_index[0].reshape(16, _NBATCH, _BSZ)
    dst3 = edge_index[1].reshape(16, _NBATCH, _BSZ)
    h0 = jnp.concatenate([pos, x], axis=1)
    h0p = jnp.pad(h0, ((0, _NP - _N), (0, 0)))
    bcol = jnp.pad(batch.astype(jnp.float32), (0, _NP - _N),
                   constant_values=float(_NUM_GRAPHS))[:, None]

    hwl0, hwr0, an20 = _stage0(h0p, W0, aS0, aD0)
    outp0, denp0 = _edge_phase(hwl0, hwr0,
                               an20[:, 0].reshape(_NP),
                               an20[:, 1].reshape(_NP), src3, dst3)
    hwl1, hwr1, an21 = _normstage(outp0, denp0, hwl0, hwr0, an20, bG0,
                                  W1, aS1, aD1)
    outp1, denp1 = _edge_phase(hwl1, hwr1,
                               an21[:, 0].reshape(_NP),
                               an21[:, 1].reshape(_NP), src3, dst3)
    return _normhead(outp1, denp1, hwl1, hwr1, an21, bG1, bcol,
                     LW0, Lb0, LW1, Lb1)


# speedup vs baseline: 27.6590x; 1.5149x over previous
import dataclasses

import jax
import jax.numpy as jnp
from jax import lax
from jax.experimental import pallas as pl
from jax.experimental.pallas import tpu as pltpu
from jax.experimental.pallas import tpu_sc as plsc

_N = 10000
_E = 320000
_D = 128
_HD = 64
_NUM_GRAPHS = 16

_NBATCH = 250
_BSZ = 80
_NP = 10240
_ROWS_PER_TILE = _NP // 16

_GRID = 8
_BLK = _NP // _GRID


def _stage0_body(h_ref, w_ref, a2_ref, hwl_ref, hwr_ref, an2_ref):
    hw = jnp.dot(h_ref[...], w_ref[...], preferred_element_type=jnp.float32)
    hwl_ref[...] = hw[:, :_HD]
    hwr_ref[...] = hw[:, _HD:]
    an2_ref[...] = jnp.dot(hw, a2_ref[...], preferred_element_type=jnp.float32)


def _stage0(h, W, aS, aD):
    a2 = jnp.stack([aS, aD], axis=1)
    return pl.pallas_call(
        _stage0_body,
        grid=(_GRID,),
        in_specs=[
            pl.BlockSpec((_BLK, _D), lambda i: (i, 0)),
            pl.BlockSpec((_D, _D), lambda i: (0, 0)),
            pl.BlockSpec((_D, 2), lambda i: (0, 0)),
        ],
        out_specs=(
            pl.BlockSpec((_BLK, _HD), lambda i: (i, 0)),
            pl.BlockSpec((_BLK, _HD), lambda i: (i, 0)),
            pl.BlockSpec((_BLK, 2), lambda i: (i, 0)),
        ),
        out_shape=(
            jax.ShapeDtypeStruct((_NP, _HD), jnp.float32),
            jax.ShapeDtypeStruct((_NP, _HD), jnp.float32),
            jax.ShapeDtypeStruct((_NP, 2), jnp.float32),
        ),
    )(h, W, a2)


_U = 80
_NU = 250


def _edge_body(hwl_hbm, hwr_hbm, ans_hbm, and_hbm, srcf_hbm,
               dst3_hbm, outp_hbm, denp_hbm,
               ans_v, and_v, src2_v, dst2_v, ex0_v, ex1_v, rows0_v, rows1_v,
               zrows_v, gsem0, gsem1, ssem0, ssem1,
               out_sh, den_sh):
    cid = lax.axis_index("c")
    sid = lax.axis_index("s")

    pltpu.sync_copy(ans_hbm, ans_v)
    pltpu.sync_copy(and_hbm, and_v)
    pltpu.sync_copy(srcf_hbm.at[sid], src2_v)
    pltpu.sync_copy(dst3_hbm.at[sid], dst2_v)

    zero16 = jnp.zeros((16,), jnp.float32)

    @pl.loop(0, 8)
    def _fill_zrows(i):
        for f in range(_HD // 16):
            zrows_v[i, pl.ds(16 * f, 16)] = zero16

    rbase = sid * _ROWS_PER_TILE

    @pl.loop(0, 80)
    def _zero_out(i):
        pltpu.sync_copy(zrows_v, out_sh.at[pl.ds(rbase + 8 * i, 8)])

    @pl.when(cid == 0)
    def _zero_den():
        for i in range(10):
            pltpu.sync_copy(zrows_v.at[0],
                            den_sh.at[pl.ds(rbase + 64 * i, 64)])

    plsc.subcore_barrier()

    def _run(hw_hbm, with_den):
        def g_issue(u, buf, sem):
            pltpu.async_copy(hw_hbm.at[src2_v.at[u]], buf, sem)

        def g_wait(buf, sem):
            pltpu.make_async_copy(
                hw_hbm.at[src2_v.at[0]], buf, sem).wait()

        def s_issue(u, buf, sem):
            pltpu.async_copy(buf, out_sh.at[dst2_v.at[u]], sem, add=True)

        def s_wait(buf, sem):
            pltpu.make_async_copy(buf, out_sh.at[dst2_v.at[0]], sem).wait()

        def exu(u, exbuf):
            for g in range(5):
                s16 = src2_v[u, pl.ds(16 * g, 16)]
                d16 = dst2_v[u, pl.ds(16 * g, 16)]
                t = (plsc.load_gather(ans_v, [s16])
                     + plsc.load_gather(and_v, [d16]))
                exbuf[pl.ds(16 * g, 16)] = jnp.exp(jnp.maximum(t, 0.2 * t))

        def scale(exbuf, buf):
            @pl.loop(0, _U // 4)
            def _s4(t):
                for q in range(4):
                    e = 4 * t + q
                    idx = jnp.full((16,), e, dtype=jnp.int32)
                    exs = plsc.load_gather(exbuf, [idx])
                    for f in range(_HD // 16):
                        sl = pl.ds(16 * f, 16)
                        buf[e, sl] = buf[e, sl] * exs

        g_issue(0, rows0_v, gsem0)
        g_issue(1, rows1_v, gsem1)

        @pl.loop(0, _NU // 2)
        def _pair(i):
            u0 = 2 * i
            u1 = 2 * i + 1
            exu(u0, ex0_v)
            exu(u1, ex1_v)
            g_wait(rows0_v, gsem0)
            scale(ex0_v, rows0_v)
            s_issue(u0, rows0_v, ssem0)
            if with_den:
                pltpu.sync_copy(ex0_v, den_sh.at[dst2_v.at[u0]], add=True)
            g_wait(rows1_v, gsem1)
            scale(ex1_v, rows1_v)
            s_issue(u1, rows1_v, ssem1)
            if with_den:
                pltpu.sync_copy(ex1_v, den_sh.at[dst2_v.at[u1]], add=True)
            s_wait(rows0_v, ssem0)

            @pl.when(i < _NU // 2 - 1)
            def _next0():
                g_issue(2 * i + 2, rows0_v, gsem0)
            s_wait(rows1_v, ssem1)

            @pl.when(i < _NU // 2 - 1)
            def _next1():
                g_issue(2 * i + 3, rows1_v, gsem1)

    @pl.when(cid == 0)
    def _run0():
        _run(hwl_hbm, True)

    @pl.when(cid == 1)
    def _run1():
        _run(hwr_hbm, False)

    plsc.subcore_barrier()

    pltpu.sync_copy(out_sh.at[pl.ds(rbase, _ROWS_PER_TILE)],
                    outp_hbm.at[cid, pl.ds(rbase, _ROWS_PER_TILE)])

    @pl.when(cid == 0)
    def _write_den():
        pltpu.sync_copy(den_sh.at[pl.ds(rbase, _ROWS_PER_TILE)],
                        denp_hbm.at[pl.ds(rbase, _ROWS_PER_TILE)])


def _edge_phase(hwl, hwr, ans, andd, srcf, dst3):
    mesh = plsc.VectorSubcoreMesh(core_axis_name="c", subcore_axis_name="s")
    cp = pltpu.CompilerParams(needs_layout_passes=False,
                              use_tc_tiling_on_sc=False)
    k = pl.kernel(
        _edge_body,
        compiler_params=cp,
        out_type=(
            jax.ShapeDtypeStruct((2, _NP, _HD), jnp.float32),
            jax.ShapeDtypeStruct((_NP,), jnp.float32),
        ),
        mesh=mesh,
        scratch_types=[
            pltpu.VMEM((_NP,), jnp.float32),
            pltpu.VMEM((_NP,), jnp.float32),
            pltpu.VMEM((_NU, _U), jnp.int32),
            pltpu.VMEM((_NU, _U), jnp.int32),
            pltpu.VMEM((_U,), jnp.float32),
            pltpu.VMEM((_U,), jnp.float32),
            pltpu.VMEM((_U, _HD), jnp.float32),
            pltpu.VMEM((_U, _HD), jnp.float32),
            pltpu.VMEM((8, _HD), jnp.float32),
            pltpu.SemaphoreType.DMA,
            pltpu.SemaphoreType.DMA,
            pltpu.SemaphoreType.DMA,
            pltpu.SemaphoreType.DMA,
            pltpu.VMEM_SHARED((_NP, _HD), jnp.float32),
            pltpu.VMEM_SHARED((_NP,), jnp.float32),
        ],
    )
    return k(hwl, hwr, ans, andd, srcf, dst3)


def _norm_h(outp_ref, denp_ref, hwl_ref, hwr_ref, an2_ref, b_ref):
    an2 = an2_ref[...]
    ex_loop = jnp.exp(jnp.maximum(an2[:, 0:1] + an2[:, 1:2],
                                  0.2 * (an2[:, 0:1] + an2[:, 1:2])))
    hw = jnp.concatenate([hwl_ref[...], hwr_ref[...]], axis=1)
    num = (jnp.concatenate([outp_ref[0], outp_ref[1]], axis=1)
           + ex_loop * hw)
    den = denp_ref[...] + ex_loop + 1e-16
    return num / den + b_ref[...]


def _normstage_body(outp_ref, denp_ref, hwl_ref, hwr_ref, an2_ref, b_ref,
                    w_ref, a2_ref, hwl2_ref, hwr2_ref, an22_ref):
    h = _norm_h(outp_ref, denp_ref, hwl_ref, hwr_ref, an2_ref, b_ref)
    hw2 = jnp.dot(h, w_ref[...], preferred_element_type=jnp.float32)
    hwl2_ref[...] = hw2[:, :_HD]
    hwr2_ref[...] = hw2[:, _HD:]
    an22_ref[...] = jnp.dot(hw2, a2_ref[...],
                            preferred_element_type=jnp.float32)


def _normstage(outp, denp, hwl, hwr, an2, b, W, aS, aD):
    a2 = jnp.stack([aS, aD], axis=1)
    return pl.pallas_call(
        _normstage_body,
        grid=(_GRID,),
        in_specs=[
            pl.BlockSpec((2, _BLK, _HD), lambda i: (0, i, 0)),
            pl.BlockSpec((_BLK, 1), lambda i: (i, 0)),
            pl.BlockSpec((_BLK, _HD), lambda i: (i, 0)),
            pl.BlockSpec((_BLK, _HD), lambda i: (i, 0)),
            pl.BlockSpec((_BLK, 2), lambda i: (i, 0)),
            pl.BlockSpec((1, _D), lambda i: (0, 0)),
            pl.BlockSpec((_D, _D), lambda i: (0, 0)),
            pl.BlockSpec((_D, 2), lambda i: (0, 0)),
        ],
        out_specs=(
            pl.BlockSpec((_BLK, _HD), lambda i: (i, 0)),
            pl.BlockSpec((_BLK, _HD), lambda i: (i, 0)),
            pl.BlockSpec((_BLK, 2), lambda i: (i, 0)),
        ),
        out_shape=(
            jax.ShapeDtypeStruct((_NP, _HD), jnp.float32),
            jax.ShapeDtypeStruct((_NP, _HD), jnp.float32),
            jax.ShapeDtypeStruct((_NP, 2), jnp.float32),
        ),
    )(outp, denp[:, None], hwl, hwr, an2, b[None, :], W, a2)


def _normhead_body(outp_ref, denp_ref, hwl_ref, hwr_ref, an2_ref, b_ref,
                   bvec_ref, lw0_ref, lb0_ref, lw1_ref, lb1_ref, o_ref,
                   sums_ref, counts_ref):
    i = pl.program_id(0)

    @pl.when(i == 0)
    def _init():
        sums_ref[...] = jnp.zeros_like(sums_ref)
        counts_ref[...] = jnp.zeros_like(counts_ref)

    h = _norm_h(outp_ref, denp_ref, hwl_ref, hwr_ref, an2_ref, b_ref)
    bcol = bvec_ref[...]
    gids = jax.lax.broadcasted_iota(
        jnp.int32, (1, _NUM_GRAPHS), 1).astype(jnp.float32)
    mask = (bcol == gids).astype(jnp.float32)
    sums_ref[...] += jax.lax.dot_general(
        mask, h, (((0,), (0,)), ((), ())),
        preferred_element_type=jnp.float32)
    counts_ref[...] += jnp.sum(mask, axis=0)[:, None]

    @pl.when(i == _GRID - 1)
    def _finish():
        pooled = sums_ref[...] / jnp.maximum(counts_ref[...], 1.0)
        y = jnp.maximum(pooled @ lw0_ref[...] + lb0_ref[...], 0.0)
        y = jnp.maximum(y @ lw1_ref[...] + lb1_ref[...], 0.0)
        o_ref[...] = y


def _normhead(outp, denp, hwl, hwr, an2, b, bcol, LW0, Lb0, LW1, Lb1):
    return pl.pallas_call(
        _normhead_body,
        grid=(_GRID,),
        in_specs=[
            pl.BlockSpec((2, _BLK, _HD), lambda i: (0, i, 0)),
            pl.BlockSpec((_BLK, 1), lambda i: (i, 0)),
            pl.BlockSpec((_BLK, _HD), lambda i: (i, 0)),
            pl.BlockSpec((_BLK, _HD), lambda i: (i, 0)),
            pl.BlockSpec((_BLK, 2), lambda i: (i, 0)),
            pl.BlockSpec((1, _D), lambda i: (0, 0)),
            pl.BlockSpec((_BLK, 1), lambda i: (i, 0)),
            pl.BlockSpec((_D, _HD), lambda i: (0, 0)),
            pl.BlockSpec((1, _HD), lambda i: (0, 0)),
            pl.BlockSpec((_HD, 10), lambda i: (0, 0)),
            pl.BlockSpec((1, 10), lambda i: (0, 0)),
        ],
        out_specs=pl.BlockSpec((_NUM_GRAPHS, 10), lambda i: (0, 0)),
        out_shape=jax.ShapeDtypeStruct((_NUM_GRAPHS, 10), jnp.float32),
        scratch_shapes=[
            pltpu.VMEM((_NUM_GRAPHS, _D), jnp.float32),
            pltpu.VMEM((_NUM_GRAPHS, 1), jnp.float32),
        ],
    )(outp, denp[:, None], hwl, hwr, an2, b[None, :],
      bcol, LW0, Lb0[None, :], LW1, Lb1[None, :])


def kernel(x, pos, edge_index, batch, W0, aS0, aD0, bG0, W1, aS1, aD1, bG1,
           LW0, Lb0, LW1, Lb1):
    srcf = edge_index[0].reshape(16, _NU, _U)
    dst3 = edge_index[1].reshape(16, _NU, _U)
    h0 = jnp.concatenate([pos, x], axis=1)
    h0p = jnp.pad(h0, ((0, _NP - _N), (0, 0)))
    bcol = jnp.pad(batch.astype(jnp.float32), (0, _NP - _N),
                   constant_values=float(_NUM_GRAPHS))[:, None]

    hwl0, hwr0, an20 = _stage0(h0p, W0, aS0, aD0)
    outp0, denp0 = _edge_phase(hwl0, hwr0,
                               an20[:, 0].reshape(_NP),
                               an20[:, 1].reshape(_NP), srcf, dst3)
    hwl1, hwr1, an21 = _normstage(outp0, denp0, hwl0, hwr0, an20, bG0,
                                  W1, aS1, aD1)
    outp1, denp1 = _edge_phase(hwl1, hwr1,
                               an21[:, 0].reshape(_NP),
                               an21[:, 1].reshape(_NP), srcf, dst3)
    return _normhead(outp1, denp1, hwl1, hwr1, an21, bG1, bcol,
                     LW0, Lb0, LW1, Lb1)


# speedup vs baseline: 38.1674x; 1.3799x over previous
import dataclasses

import jax
import jax.numpy as jnp
from jax import lax
from jax.experimental import pallas as pl
from jax.experimental.pallas import tpu as pltpu
from jax.experimental.pallas import tpu_sc as plsc

_N = 10000
_E = 320000
_D = 128
_HD = 64
_NUM_GRAPHS = 16

_NBATCH = 250
_BSZ = 80
_NP = 10240
_ROWS_PER_TILE = _NP // 16

_GRID = 8
_BLK = _NP // _GRID


def _stage0_body(h_ref, w_ref, a2_ref, hwl_ref, hwr_ref, an2_ref):
    hw = jnp.dot(h_ref[...], w_ref[...], preferred_element_type=jnp.float32)
    hwl_ref[...] = hw[:, :_HD]
    hwr_ref[...] = hw[:, _HD:]
    an2_ref[...] = jnp.dot(hw, a2_ref[...], preferred_element_type=jnp.float32)


def _stage0(h, W, aS, aD):
    a2 = jnp.stack([aS, aD], axis=1)
    return pl.pallas_call(
        _stage0_body,
        grid=(_GRID,),
        in_specs=[
            pl.BlockSpec((_BLK, _D), lambda i: (i, 0)),
            pl.BlockSpec((_D, _D), lambda i: (0, 0)),
            pl.BlockSpec((_D, 2), lambda i: (0, 0)),
        ],
        out_specs=(
            pl.BlockSpec((_BLK, _HD), lambda i: (i, 0)),
            pl.BlockSpec((_BLK, _HD), lambda i: (i, 0)),
            pl.BlockSpec((_BLK, 2), lambda i: (i, 0)),
        ),
        out_shape=(
            jax.ShapeDtypeStruct((_NP, _HD), jnp.float32),
            jax.ShapeDtypeStruct((_NP, _HD), jnp.float32),
            jax.ShapeDtypeStruct((_NP, 2), jnp.float32),
        ),
    )(h, W, a2)


_U = 80
_NU = 250


def _edge_body(hwl_hbm, hwr_hbm, ans_hbm, and_hbm, srcf_hbm,
               dst3_hbm, outp_hbm, denp_hbm,
               ans_v, and_v, src2_v, dst2_v, ex0_v, ex1_v, rows0_v, rows1_v,
               zrows_v, gsem0, gsem1, ssem0, ssem1, dsem,
               out_sh, den_sh):
    cid = lax.axis_index("c")
    sid = lax.axis_index("s")

    pltpu.sync_copy(ans_hbm, ans_v)
    pltpu.sync_copy(and_hbm, and_v)
    pltpu.sync_copy(srcf_hbm.at[sid], src2_v)
    pltpu.sync_copy(dst3_hbm.at[sid], dst2_v)

    zero16 = jnp.zeros((16,), jnp.float32)

    @pl.loop(0, 8)
    def _fill_zrows(i):
        for f in range(_HD // 16):
            zrows_v[i, pl.ds(16 * f, 16)] = zero16

    rbase = sid * _ROWS_PER_TILE

    @pl.loop(0, 80)
    def _zero_out(i):
        pltpu.sync_copy(zrows_v, out_sh.at[pl.ds(rbase + 8 * i, 8)])

    @pl.when(cid == 0)
    def _zero_den():
        for i in range(10):
            pltpu.sync_copy(zrows_v.at[0],
                            den_sh.at[pl.ds(rbase + 64 * i, 64)])

    plsc.subcore_barrier()

    def _run(hw_hbm, with_den):
        def g_issue(u, buf, sem):
            pltpu.async_copy(hw_hbm.at[src2_v.at[u]], buf, sem)

        def g_wait(buf, sem):
            pltpu.make_async_copy(
                hw_hbm.at[src2_v.at[0]], buf, sem).wait()

        def s_issue(u, buf, sem):
            pltpu.async_copy(buf, out_sh.at[dst2_v.at[u]], sem, add=True)

        def s_wait(buf, sem):
            pltpu.make_async_copy(buf, out_sh.at[dst2_v.at[0]], sem).wait()

        def d_issue(u, exbuf):
            pltpu.async_copy(exbuf, den_sh.at[dst2_v.at[u]], dsem, add=True)

        def d_wait():
            pltpu.make_async_copy(ex0_v, den_sh.at[dst2_v.at[0]],
                                  dsem).wait()

        def exu(u, exbuf):
            for g in range(5):
                s16 = src2_v[u, pl.ds(16 * g, 16)]
                d16 = dst2_v[u, pl.ds(16 * g, 16)]
                t = (plsc.load_gather(ans_v, [s16])
                     + plsc.load_gather(and_v, [d16]))
                exbuf[pl.ds(16 * g, 16)] = jnp.exp(jnp.maximum(t, 0.2 * t))

        def scale(exbuf, buf):
            @plsc.parallel_loop(0, _U // 4, 1, unroll=2)
            def _s4(t):
                for q in range(4):
                    e = 4 * t + q
                    idx = jnp.full((16,), e, dtype=jnp.int32)
                    exs = plsc.load_gather(exbuf, [idx])
                    for f in range(_HD // 16):
                        sl = pl.ds(16 * f, 16)
                        buf[e, sl] = buf[e, sl] * exs

        g_issue(0, rows0_v, gsem0)
        g_issue(1, rows1_v, gsem1)

        @pl.loop(0, _NU // 2)
        def _pair(i):
            u0 = 2 * i
            u1 = 2 * i + 1
            if with_den:
                @pl.when(i > 0)
                def _dwtop():
                    d_wait()
                    d_wait()
            exu(u0, ex0_v)
            exu(u1, ex1_v)
            g_wait(rows0_v, gsem0)
            scale(ex0_v, rows0_v)
            s_issue(u0, rows0_v, ssem0)
            if with_den:
                d_issue(u0, ex0_v)
            g_wait(rows1_v, gsem1)
            scale(ex1_v, rows1_v)
            s_issue(u1, rows1_v, ssem1)
            if with_den:
                d_issue(u1, ex1_v)
            s_wait(rows0_v, ssem0)

            @pl.when(i < _NU // 2 - 1)
            def _next0():
                g_issue(2 * i + 2, rows0_v, gsem0)
            s_wait(rows1_v, ssem1)

            @pl.when(i < _NU // 2 - 1)
            def _next1():
                g_issue(2 * i + 3, rows1_v, gsem1)

        if with_den:
            d_wait()
            d_wait()

    @pl.when(cid == 0)
    def _run0():
        _run(hwl_hbm, True)

    @pl.when(cid == 1)
    def _run1():
        _run(hwr_hbm, False)

    plsc.subcore_barrier()

    pltpu.sync_copy(out_sh.at[pl.ds(rbase, _ROWS_PER_TILE)],
                    outp_hbm.at[cid, pl.ds(rbase, _ROWS_PER_TILE)])

    @pl.when(cid == 0)
    def _write_den():
        pltpu.sync_copy(den_sh.at[pl.ds(rbase, _ROWS_PER_TILE)],
                        denp_hbm.at[pl.ds(rbase, _ROWS_PER_TILE)])


def _edge_phase(hwl, hwr, ans, andd, srcf, dst3):
    mesh = plsc.VectorSubcoreMesh(core_axis_name="c", subcore_axis_name="s")
    cp = pltpu.CompilerParams(needs_layout_passes=False,
                              use_tc_tiling_on_sc=False)
    k = pl.kernel(
        _edge_body,
        compiler_params=cp,
        out_type=(
            jax.ShapeDtypeStruct((2, _NP, _HD), jnp.float32),
            jax.ShapeDtypeStruct((_NP,), jnp.float32),
        ),
        mesh=mesh,
        scratch_types=[
            pltpu.VMEM((_NP,), jnp.float32),
            pltpu.VMEM((_NP,), jnp.float32),
            pltpu.VMEM((_NU, _U), jnp.int32),
            pltpu.VMEM((_NU, _U), jnp.int32),
            pltpu.VMEM((_U,), jnp.float32),
            pltpu.VMEM((_U,), jnp.float32),
            pltpu.VMEM((_U, _HD), jnp.float32),
            pltpu.VMEM((_U, _HD), jnp.float32),
            pltpu.VMEM((8, _HD), jnp.float32),
            pltpu.SemaphoreType.DMA,
            pltpu.SemaphoreType.DMA,
            pltpu.SemaphoreType.DMA,
            pltpu.SemaphoreType.DMA,
            pltpu.SemaphoreType.DMA,
            pltpu.VMEM_SHARED((_NP, _HD), jnp.float32),
            pltpu.VMEM_SHARED((_NP,), jnp.float32),
        ],
    )
    return k(hwl, hwr, ans, andd, srcf, dst3)


def _norm_h(outp_ref, denp_ref, hwl_ref, hwr_ref, an2_ref, b_ref):
    an2 = an2_ref[...]
    ex_loop = jnp.exp(jnp.maximum(an2[:, 0:1] + an2[:, 1:2],
                                  0.2 * (an2[:, 0:1] + an2[:, 1:2])))
    hw = jnp.concatenate([hwl_ref[...], hwr_ref[...]], axis=1)
    num = (jnp.concatenate([outp_ref[0], outp_ref[1]], axis=1)
           + ex_loop * hw)
    den = denp_ref[...] + ex_loop + 1e-16
    return num / den + b_ref[...]


def _normstage_body(outp_ref, denp_ref, hwl_ref, hwr_ref, an2_ref, b_ref,
                    w_ref, a2_ref, hwl2_ref, hwr2_ref, an22_ref):
    h = _norm_h(outp_ref, denp_ref, hwl_ref, hwr_ref, an2_ref, b_ref)
    hw2 = jnp.dot(h, w_ref[...], preferred_element_type=jnp.float32)
    hwl2_ref[...] = hw2[:, :_HD]
    hwr2_ref[...] = hw2[:, _HD:]
    an22_ref[...] = jnp.dot(hw2, a2_ref[...],
                            preferred_element_type=jnp.float32)


def _normstage(outp, denp, hwl, hwr, an2, b, W, aS, aD):
    a2 = jnp.stack([aS, aD], axis=1)
    return pl.pallas_call(
        _normstage_body,
        grid=(_GRID,),
        in_specs=[
            pl.BlockSpec((2, _BLK, _HD), lambda i: (0, i, 0)),
            pl.BlockSpec((_BLK, 1), lambda i: (i, 0)),
            pl.BlockSpec((_BLK, _HD), lambda i: (i, 0)),
            pl.BlockSpec((_BLK, _HD), lambda i: (i, 0)),
            pl.BlockSpec((_BLK, 2), lambda i: (i, 0)),
            pl.BlockSpec((1, _D), lambda i: (0, 0)),
            pl.BlockSpec((_D, _D), lambda i: (0, 0)),
            pl.BlockSpec((_D, 2), lambda i: (0, 0)),
        ],
        out_specs=(
            pl.BlockSpec((_BLK, _HD), lambda i: (i, 0)),
            pl.BlockSpec((_BLK, _HD), lambda i: (i, 0)),
            pl.BlockSpec((_BLK, 2), lambda i: (i, 0)),
        ),
        out_shape=(
            jax.ShapeDtypeStruct((_NP, _HD), jnp.float32),
            jax.ShapeDtypeStruct((_NP, _HD), jnp.float32),
            jax.ShapeDtypeStruct((_NP, 2), jnp.float32),
        ),
    )(outp, denp[:, None], hwl, hwr, an2, b[None, :], W, a2)


def _normhead_body(outp_ref, denp_ref, hwl_ref, hwr_ref, an2_ref, b_ref,
                   bvec_ref, lw0_ref, lb0_ref, lw1_ref, lb1_ref, o_ref,
                   sums_ref, counts_ref):
    i = pl.program_id(0)

    @pl.when(i == 0)
    def _init():
        sums_ref[...] = jnp.zeros_like(sums_ref)
        counts_ref[...] = jnp.zeros_like(counts_ref)

    h = _norm_h(outp_ref, denp_ref, hwl_ref, hwr_ref, an2_ref, b_ref)
    bcol = bvec_ref[...]
    gids = jax.lax.broadcasted_iota(
        jnp.int32, (1, _NUM_GRAPHS), 1).astype(jnp.float32)
    mask = (bcol == gids).astype(jnp.float32)
    sums_ref[...] += jax.lax.dot_general(
        mask, h, (((0,), (0,)), ((), ())),
        preferred_element_type=jnp.float32)
    counts_ref[...] += jnp.sum(mask, axis=0)[:, None]

    @pl.when(i == _GRID - 1)
    def _finish():
        pooled = sums_ref[...] / jnp.maximum(counts_ref[...], 1.0)
        y = jnp.maximum(pooled @ lw0_ref[...] + lb0_ref[...], 0.0)
        y = jnp.maximum(y @ lw1_ref[...] + lb1_ref[...], 0.0)
        o_ref[...] = y


def _normhead(outp, denp, hwl, hwr, an2, b, bcol, LW0, Lb0, LW1, Lb1):
    return pl.pallas_call(
        _normhead_body,
        grid=(_GRID,),
        in_specs=[
            pl.BlockSpec((2, _BLK, _HD), lambda i: (0, i, 0)),
            pl.BlockSpec((_BLK, 1), lambda i: (i, 0)),
            pl.BlockSpec((_BLK, _HD), lambda i: (i, 0)),
            pl.BlockSpec((_BLK, _HD), lambda i: (i, 0)),
            pl.BlockSpec((_BLK, 2), lambda i: (i, 0)),
            pl.BlockSpec((1, _D), lambda i: (0, 0)),
            pl.BlockSpec((_BLK, 1), lambda i: (i, 0)),
            pl.BlockSpec((_D, _HD), lambda i: (0, 0)),
            pl.BlockSpec((1, _HD), lambda i: (0, 0)),
            pl.BlockSpec((_HD, 10), lambda i: (0, 0)),
            pl.BlockSpec((1, 10), lambda i: (0, 0)),
        ],
        out_specs=pl.BlockSpec((_NUM_GRAPHS, 10), lambda i: (0, 0)),
        out_shape=jax.ShapeDtypeStruct((_NUM_GRAPHS, 10), jnp.float32),
        scratch_shapes=[
            pltpu.VMEM((_NUM_GRAPHS, _D), jnp.float32),
            pltpu.VMEM((_NUM_GRAPHS, 1), jnp.float32),
        ],
    )(outp, denp[:, None], hwl, hwr, an2, b[None, :],
      bcol, LW0, Lb0[None, :], LW1, Lb1[None, :])


def kernel(x, pos, edge_index, batch, W0, aS0, aD0, bG0, W1, aS1, aD1, bG1,
           LW0, Lb0, LW1, Lb1):
    srcf = edge_index[0].reshape(16, _NU, _U)
    dst3 = edge_index[1].reshape(16, _NU, _U)
    h0 = jnp.concatenate([pos, x], axis=1)
    h0p = jnp.pad(h0, ((0, _NP - _N), (0, 0)))
    bcol = jnp.pad(batch.astype(jnp.float32), (0, _NP - _N),
                   constant_values=float(_NUM_GRAPHS))[:, None]

    hwl0, hwr0, an20 = _stage0(h0p, W0, aS0, aD0)
    outp0, denp0 = _edge_phase(hwl0, hwr0,
                               an20[:, 0].reshape(_NP),
                               an20[:, 1].reshape(_NP), srcf, dst3)
    hwl1, hwr1, an21 = _normstage(outp0, denp0, hwl0, hwr0, an20, bG0,
                                  W1, aS1, aD1)
    outp1, denp1 = _edge_phase(hwl1, hwr1,
                               an21[:, 0].reshape(_NP),
                               an21[:, 1].reshape(_NP), srcf, dst3)
    return _normhead(outp1, denp1, hwl1, hwr1, an21, bG1, bcol,
                     LW0, Lb0, LW1, Lb1)


# speedup vs baseline: 40.5031x; 1.0612x over previous
import dataclasses

import jax
import jax.numpy as jnp
from jax import lax
from jax.experimental import pallas as pl
from jax.experimental.pallas import tpu as pltpu
from jax.experimental.pallas import tpu_sc as plsc

_N = 10000
_E = 320000
_D = 128
_HD = 64
_NUM_GRAPHS = 16

_NBATCH = 250
_BSZ = 80
_NP = 10240
_ROWS_PER_TILE = _NP // 16

_GRID = 8
_BLK = _NP // _GRID


def _stage0_body(h_ref, w_ref, a2_ref, hwl_ref, hwr_ref, an2_ref):
    hw = jnp.dot(h_ref[...], w_ref[...], preferred_element_type=jnp.float32)
    hwl_ref[...] = hw[:, :_HD]
    hwr_ref[...] = hw[:, _HD:]
    an2_ref[...] = jnp.dot(hw, a2_ref[...], preferred_element_type=jnp.float32)


def _stage0(h, W, aS, aD):
    a2 = jnp.stack([aS, aD], axis=1)
    return pl.pallas_call(
        _stage0_body,
        grid=(_GRID,),
        in_specs=[
            pl.BlockSpec((_BLK, _D), lambda i: (i, 0)),
            pl.BlockSpec((_D, _D), lambda i: (0, 0)),
            pl.BlockSpec((_D, 2), lambda i: (0, 0)),
        ],
        out_specs=(
            pl.BlockSpec((_BLK, _HD), lambda i: (i, 0)),
            pl.BlockSpec((_BLK, _HD), lambda i: (i, 0)),
            pl.BlockSpec((_BLK, 2), lambda i: (i, 0)),
        ),
        out_shape=(
            jax.ShapeDtypeStruct((_NP, _HD), jnp.float32),
            jax.ShapeDtypeStruct((_NP, _HD), jnp.float32),
            jax.ShapeDtypeStruct((_NP, 2), jnp.float32),
        ),
    )(h, W, a2)


_U = 80
_NU = 250


def _edge_body(hwl_hbm, hwr_hbm, ans_hbm, and_hbm, srcf_hbm,
               dst3_hbm, outp_hbm, denp_hbm,
               ans_v, and_v, src2_v, dst2_v, ex0_v, ex1_v, ex2_v, ex3_v,
               rows0_v, rows1_v, rows2_v, rows3_v,
               zrows_v, gsem0, gsem1, gsem2, gsem3,
               ssem0, ssem1, ssem2, ssem3, dsem,
               out_sh, den_sh):
    cid = lax.axis_index("c")
    sid = lax.axis_index("s")

    pltpu.sync_copy(ans_hbm, ans_v)
    pltpu.sync_copy(and_hbm, and_v)
    pltpu.sync_copy(srcf_hbm.at[sid], src2_v)
    pltpu.sync_copy(dst3_hbm.at[sid], dst2_v)

    zero16 = jnp.zeros((16,), jnp.float32)

    @pl.loop(0, 8)
    def _fill_zrows(i):
        for f in range(_HD // 16):
            zrows_v[i, pl.ds(16 * f, 16)] = zero16

    rbase = sid * _ROWS_PER_TILE

    @pl.loop(0, 80)
    def _zero_out(i):
        pltpu.sync_copy(zrows_v, out_sh.at[pl.ds(rbase + 8 * i, 8)])

    @pl.when(cid == 0)
    def _zero_den():
        for i in range(10):
            pltpu.sync_copy(zrows_v.at[0],
                            den_sh.at[pl.ds(rbase + 64 * i, 64)])

    plsc.subcore_barrier()

    def _run(hw_hbm, with_den):
        def g_issue(u, buf, sem):
            pltpu.async_copy(hw_hbm.at[src2_v.at[u]], buf, sem)

        def g_wait(buf, sem):
            pltpu.make_async_copy(
                hw_hbm.at[src2_v.at[0]], buf, sem).wait()

        def s_issue(u, buf, sem):
            pltpu.async_copy(buf, out_sh.at[dst2_v.at[u]], sem, add=True)

        def s_wait(buf, sem):
            pltpu.make_async_copy(buf, out_sh.at[dst2_v.at[0]], sem).wait()

        def d_issue(u, exbuf):
            pltpu.async_copy(exbuf, den_sh.at[dst2_v.at[u]], dsem, add=True)

        def d_wait():
            pltpu.make_async_copy(ex0_v, den_sh.at[dst2_v.at[0]],
                                  dsem).wait()

        def exu(u, exbuf):
            for g in range(5):
                s16 = src2_v[u, pl.ds(16 * g, 16)]
                d16 = dst2_v[u, pl.ds(16 * g, 16)]
                t = (plsc.load_gather(ans_v, [s16])
                     + plsc.load_gather(and_v, [d16]))
                exbuf[pl.ds(16 * g, 16)] = jnp.exp(jnp.maximum(t, 0.2 * t))

        def scale(exbuf, buf):
            @plsc.parallel_loop(0, _U // 4, 1, unroll=2)
            def _s4(t):
                for q in range(4):
                    e = 4 * t + q
                    idx = jnp.full((16,), e, dtype=jnp.int32)
                    exs = plsc.load_gather(exbuf, [idx])
                    for f in range(_HD // 16):
                        sl = pl.ds(16 * f, 16)
                        buf[e, sl] = buf[e, sl] * exs

        bufs = ((rows0_v, ex0_v, gsem0, ssem0),
                (rows1_v, ex1_v, gsem1, ssem1),
                (rows2_v, ex2_v, gsem2, ssem2),
                (rows3_v, ex3_v, gsem3, ssem3))
        for b in range(4):
            g_issue(b, bufs[b][0], bufs[b][2])

        nq = _NU // 4

        @pl.loop(0, nq)
        def _quad(i):
            if with_den:
                @pl.when(i > 0)
                def _dwtop():
                    for _ in range(4):
                        d_wait()
            for b in range(4):
                buf, exb, gsem, ssem = bufs[b]
                u = 4 * i + b
                exu(u, exb)
                g_wait(buf, gsem)
                scale(exb, buf)
                s_issue(u, buf, ssem)
                if with_den:
                    d_issue(u, exb)
            for b in range(4):
                buf, exb, gsem, ssem = bufs[b]
                s_wait(buf, ssem)

                @pl.when(i < nq - 1)
                def _nextg():
                    g_issue(4 * i + 4 + b, buf, gsem)

        if with_den:
            for _ in range(4):
                d_wait()

    @pl.when(cid == 0)
    def _run0():
        _run(hwl_hbm, True)

    @pl.when(cid == 1)
    def _run1():
        _run(hwr_hbm, False)

    plsc.subcore_barrier()

    pltpu.sync_copy(out_sh.at[pl.ds(rbase, _ROWS_PER_TILE)],
                    outp_hbm.at[cid, pl.ds(rbase, _ROWS_PER_TILE)])

    @pl.when(cid == 0)
    def _write_den():
        pltpu.sync_copy(den_sh.at[pl.ds(rbase, _ROWS_PER_TILE)],
                        denp_hbm.at[pl.ds(rbase, _ROWS_PER_TILE)])


def _edge_phase(hwl, hwr, ans, andd, srcf, dst3):
    mesh = plsc.VectorSubcoreMesh(core_axis_name="c", subcore_axis_name="s")
    cp = pltpu.CompilerParams(needs_layout_passes=False,
                              use_tc_tiling_on_sc=False)
    k = pl.kernel(
        _edge_body,
        compiler_params=cp,
        out_type=(
            jax.ShapeDtypeStruct((2, _NP, _HD), jnp.float32),
            jax.ShapeDtypeStruct((_NP,), jnp.float32),
        ),
        mesh=mesh,
        scratch_types=[
            pltpu.VMEM((_NP,), jnp.float32),
            pltpu.VMEM((_NP,), jnp.float32),
            pltpu.VMEM((_NU, _U), jnp.int32),
            pltpu.VMEM((_NU, _U), jnp.int32),
            pltpu.VMEM((_U,), jnp.float32),
            pltpu.VMEM((_U,), jnp.float32),
            pltpu.VMEM((_U,), jnp.float32),
            pltpu.VMEM((_U,), jnp.float32),
            pltpu.VMEM((_U, _HD), jnp.float32),
            pltpu.VMEM((_U, _HD), jnp.float32),
            pltpu.VMEM((_U, _HD), jnp.float32),
            pltpu.VMEM((_U, _HD), jnp.float32),
            pltpu.VMEM((8, _HD), jnp.float32),
            pltpu.SemaphoreType.DMA,
            pltpu.SemaphoreType.DMA,
            pltpu.SemaphoreType.DMA,
            pltpu.SemaphoreType.DMA,
            pltpu.SemaphoreType.DMA,
            pltpu.SemaphoreType.DMA,
            pltpu.SemaphoreType.DMA,
            pltpu.SemaphoreType.DMA,
            pltpu.SemaphoreType.DMA,
            pltpu.VMEM_SHARED((_NP, _HD), jnp.float32),
            pltpu.VMEM_SHARED((_NP,), jnp.float32),
        ],
    )
    return k(hwl, hwr, ans, andd, srcf, dst3)


def _norm_h(outp_ref, denp_ref, hwl_ref, hwr_ref, an2_ref, b_ref):
    an2 = an2_ref[...]
    ex_loop = jnp.exp(jnp.maximum(an2[:, 0:1] + an2[:, 1:2],
                                  0.2 * (an2[:, 0:1] + an2[:, 1:2])))
    hw = jnp.concatenate([hwl_ref[...], hwr_ref[...]], axis=1)
    num = (jnp.concatenate([outp_ref[0], outp_ref[1]], axis=1)
           + ex_loop * hw)
    den = denp_ref[...] + ex_loop + 1e-16
    return num / den + b_ref[...]


def _normstage_body(outp_ref, denp_ref, hwl_ref, hwr_ref, an2_ref, b_ref,
                    w_ref, a2_ref, hwl2_ref, hwr2_ref, an22_ref):
    h = _norm_h(outp_ref, denp_ref, hwl_ref, hwr_ref, an2_ref, b_ref)
    hw2 = jnp.dot(h, w_ref[...], preferred_element_type=jnp.float32)
    hwl2_ref[...] = hw2[:, :_HD]
    hwr2_ref[...] = hw2[:, _HD:]
    an22_ref[...] = jnp.dot(hw2, a2_ref[...],
                            preferred_element_type=jnp.float32)


def _normstage(outp, denp, hwl, hwr, an2, b, W, aS, aD):
    a2 = jnp.stack([aS, aD], axis=1)
    return pl.pallas_call(
        _normstage_body,
        grid=(_GRID,),
        in_specs=[
            pl.BlockSpec((2, _BLK, _HD), lambda i: (0, i, 0)),
            pl.BlockSpec((_BLK, 1), lambda i: (i, 0)),
            pl.BlockSpec((_BLK, _HD), lambda i: (i, 0)),
            pl.BlockSpec((_BLK, _HD), lambda i: (i, 0)),
            pl.BlockSpec((_BLK, 2), lambda i: (i, 0)),
            pl.BlockSpec((1, _D), lambda i: (0, 0)),
            pl.BlockSpec((_D, _D), lambda i: (0, 0)),
            pl.BlockSpec((_D, 2), lambda i: (0, 0)),
        ],
        out_specs=(
            pl.BlockSpec((_BLK, _HD), lambda i: (i, 0)),
            pl.BlockSpec((_BLK, _HD), lambda i: (i, 0)),
            pl.BlockSpec((_BLK, 2), lambda i: (i, 0)),
        ),
        out_shape=(
            jax.ShapeDtypeStruct((_NP, _HD), jnp.float32),
            jax.ShapeDtypeStruct((_NP, _HD), jnp.float32),
            jax.ShapeDtypeStruct((_NP, 2), jnp.float32),
        ),
    )(outp, denp[:, None], hwl, hwr, an2, b[None, :], W, a2)


def _normhead_body(outp_ref, denp_ref, hwl_ref, hwr_ref, an2_ref, b_ref,
                   bvec_ref, lw0_ref, lb0_ref, lw1_ref, lb1_ref, o_ref,
                   sums_ref, counts_ref):
    i = pl.program_id(0)

    @pl.when(i == 0)
    def _init():
        sums_ref[...] = jnp.zeros_like(sums_ref)
        counts_ref[...] = jnp.zeros_like(counts_ref)

    h = _norm_h(outp_ref, denp_ref, hwl_ref, hwr_ref, an2_ref, b_ref)
    bcol = bvec_ref[...]
    gids = jax.lax.broadcasted_iota(
        jnp.int32, (1, _NUM_GRAPHS), 1).astype(jnp.float32)
    mask = (bcol == gids).astype(jnp.float32)
    sums_ref[...] += jax.lax.dot_general(
        mask, h, (((0,), (0,)), ((), ())),
        preferred_element_type=jnp.float32)
    counts_ref[...] += jnp.sum(mask, axis=0)[:, None]

    @pl.when(i == _GRID - 1)
    def _finish():
        pooled = sums_ref[...] / jnp.maximum(counts_ref[...], 1.0)
        y = jnp.maximum(pooled @ lw0_ref[...] + lb0_ref[...], 0.0)
        y = jnp.maximum(y @ lw1_ref[...] + lb1_ref[...], 0.0)
        o_ref[...] = y


def _normhead(outp, denp, hwl, hwr, an2, b, bcol, LW0, Lb0, LW1, Lb1):
    return pl.pallas_call(
        _normhead_body,
        grid=(_GRID,),
        in_specs=[
            pl.BlockSpec((2, _BLK, _HD), lambda i: (0, i, 0)),
            pl.BlockSpec((_BLK, 1), lambda i: (i, 0)),
            pl.BlockSpec((_BLK, _HD), lambda i: (i, 0)),
            pl.BlockSpec((_BLK, _HD), lambda i: (i, 0)),
            pl.BlockSpec((_BLK, 2), lambda i: (i, 0)),
            pl.BlockSpec((1, _D), lambda i: (0, 0)),
            pl.BlockSpec((_BLK, 1), lambda i: (i, 0)),
            pl.BlockSpec((_D, _HD), lambda i: (0, 0)),
            pl.BlockSpec((1, _HD), lambda i: (0, 0)),
            pl.BlockSpec((_HD, 10), lambda i: (0, 0)),
            pl.BlockSpec((1, 10), lambda i: (0, 0)),
        ],
        out_specs=pl.BlockSpec((_NUM_GRAPHS, 10), lambda i: (0, 0)),
        out_shape=jax.ShapeDtypeStruct((_NUM_GRAPHS, 10), jnp.float32),
        scratch_shapes=[
            pltpu.VMEM((_NUM_GRAPHS, _D), jnp.float32),
            pltpu.VMEM((_NUM_GRAPHS, 1), jnp.float32),
        ],
    )(outp, denp[:, None], hwl, hwr, an2, b[None, :],
      bcol, LW0, Lb0[None, :], LW1, Lb1[None, :])


def kernel(x, pos, edge_index, batch, W0, aS0, aD0, bG0, W1, aS1, aD1, bG1,
           LW0, Lb0, LW1, Lb1):
    srcf = edge_index[0].reshape(16, _NU, _U)
    dst3 = edge_index[1].reshape(16, _NU, _U)
    h0 = jnp.concatenate([pos, x], axis=1)
    h0p = jnp.pad(h0, ((0, _NP - _N), (0, 0)))
    bcol = jnp.pad(batch.astype(jnp.float32), (0, _NP - _N),
                   constant_values=float(_NUM_GRAPHS))[:, None]

    hwl0, hwr0, an20 = _stage0(h0p, W0, aS0, aD0)
    outp0, denp0 = _edge_phase(hwl0, hwr0,
                               an20[:, 0].reshape(_NP),
                               an20[:, 1].reshape(_NP), srcf, dst3)
    hwl1, hwr1, an21 = _normstage(outp0, denp0, hwl0, hwr0, an20, bG0,
                                  W1, aS1, aD1)
    outp1, denp1 = _edge_phase(hwl1, hwr1,
                               an21[:, 0].reshape(_NP),
                               an21[:, 1].reshape(_NP), srcf, dst3)
    return _normhead(outp1, denp1, hwl1, hwr1, an21, bG1, bcol,
                     LW0, Lb0, LW1, Lb1)
